# baseline (speedup 1.0000x reference)
import functools

import jax
import jax.numpy as jnp
from jax import lax
from jax.experimental import pallas as pl
from jax.experimental.pallas import tpu as pltpu
from jax.experimental.pallas import tpu_sc as plsc

_NC = 2
_NS = 16
_NW = _NC * _NS
_L = 16

_N_ROWS = 128
_T = 32768
_K = _T // 8
_ROWS_PER_W = _N_ROWS // _NW

_B1_BITS = 12
_B2_BITS = 9
_B3_BITS = 9
_NB1 = 1 << _B1_BITS
_NB2 = 1 << _B2_BITS
_NB3 = 1 << _B3_BITS


def _walk(loadc, loads, nchunks, tau):
    iota = lax.iota(jnp.int32, _L)

    def body(i, carry):
        found, run_c, run_s, bst, pst, spst, cst, sst = carry
        cc = loadc(i)
        ss = loads(i)
        tot_c = jnp.sum(cc)
        tot_s = jnp.sum(ss)
        cross = jnp.logical_and(found == 0, run_c + tot_c > tau)

        def heavy(op):
            cc, ss, run_c, run_s, _f, _b, _p, _sp, _c, _s = op
            ic = plsc.cumsum(cc)
            p_lane = run_c + ic - cc
            m = jnp.logical_and(p_lane <= tau, tau < p_lane + cc)
            fs = plsc.cumsum(ss)
            sp_lane = run_s + fs - ss
            zi = jnp.zeros((_L,), jnp.int32)
            zf = jnp.zeros((_L,), jnp.float32)
            b = i * _L + jnp.sum(jnp.where(m, iota, zi))
            p = jnp.sum(jnp.where(m, p_lane, zi))
            sp = jnp.sum(jnp.where(m, sp_lane, zf))
            c = jnp.sum(jnp.where(m, cc, zi))
            s = jnp.sum(jnp.where(m, ss, zf))
            return (jnp.int32(1), b, p, sp, c, s)

        def light(op):
            _cc, _ss, _rc, _rs, f, b, p, sp, c, s = op
            return (f, b, p, sp, c, s)

        found, bst, pst, spst, cst, sst = lax.cond(
            cross, heavy, light,
            (cc, ss, run_c, run_s, found, bst, pst, spst, cst, sst))
        return (found, run_c + tot_c, run_s + tot_s, bst, pst, spst, cst, sst)

    z = (jnp.int32(0), jnp.int32(0), jnp.float32(0.0), jnp.int32(0),
         jnp.int32(0), jnp.float32(0.0), jnp.int32(0), jnp.float32(0.0))
    out = lax.fori_loop(0, nchunks, body, z)
    return out[3], out[4], out[5], out[6], out[7]


def _sc_loss_kernel(att_hbm, out_hbm, row_v, cnt1, sum1, h2c, h2s, h3c,
                    outbuf):
    wid = lax.axis_index("s") * _NC + lax.axis_index("c")
    iota = lax.iota(jnp.int32, _L)
    ones_i = jnp.ones((_L,), jnp.int32)
    zeros_i = jnp.zeros((_L,), jnp.int32)
    zeros_f = jnp.zeros((_L,), jnp.float32)
    kk = jnp.int32(_K)

    def row_body(r, acc):
        row = wid * _ROWS_PER_W + r
        pltpu.sync_copy(att_hbm.at[row], row_v)

        def z1(i, c):
            cnt1[pl.ds(i * _L, _L)] = zeros_i
            sum1[pl.ds(i * _L, _L)] = zeros_f
            return c
        lax.fori_loop(0, _NB1 // _L, z1, 0)

        def z2(i, c):
            h2c[pl.ds(i * _L, _L)] = zeros_i
            h2s[pl.ds(i * _L, _L)] = zeros_f
            h3c[pl.ds(i * _L, _L)] = zeros_i
            return c
        lax.fori_loop(0, (2 * _NB2) // _L, z2, 0)

        def p1(j, sacc):
            v = row_v[pl.ds(j * _L, _L)]
            p = plsc.bitcast(v, jnp.int32)
            b1 = (p >> (30 - _B1_BITS))
            plsc.addupdate_scatter(cnt1, [b1], ones_i)
            plsc.addupdate_scatter(sum1, [b1], v)
            return sacc + v
        svec = lax.fori_loop(0, _T // _L, p1, zeros_f)
        total_sum = jnp.sum(svec)

        l1c = lambda i: cnt1[pl.ds(i * _L, _L)]
        l1s = lambda i: sum1[pl.ds(i * _L, _L)]
        b1_hi, p_hi, sp_hi, c_hi, s_hi = _walk(
            l1c, l1s, _NB1 // _L, jnp.int32(_T - _K))
        b1_lo, p_lo, sp_lo, c_lo, s_lo = _walk(
            l1c, l1s, _NB1 // _L, jnp.int32(_K - 1))

        top_inc = total_sum - sp_hi - s_hi
        rk_hi = kk - (jnp.int32(_T) - p_hi - c_hi)
        bot_inc = sp_lo
        rk_lo = kk - p_lo
        same1 = b1_hi == b1_lo

        def p2(j, c):
            v = row_v[pl.ds(j * _L, _L)]
            p = plsc.bitcast(v, jnp.int32)
            b1 = (p >> (30 - _B1_BITS))
            in_hi = b1 == b1_hi
            in_lo = b1 == b1_lo
            b2 = (p >> _B3_BITS) & (_NB2 - 1)
            use_off = jnp.logical_and(in_lo, jnp.logical_not(same1))
            idx = b2 + jnp.where(use_off, _NB2, 0)
            msk = jnp.logical_or(in_hi, in_lo)
            plsc.addupdate_scatter(h2c, [idx], ones_i, mask=msk)
            plsc.addupdate_scatter(h2s, [idx], v, mask=msk)
            return c
        lax.fori_loop(0, _T // _L, p2, 0)

        off_lo = jnp.where(same1, 0, _NB2)
        l2c_hi = lambda i: h2c[pl.ds(i * _L, _L)]
        l2s_hi = lambda i: h2s[pl.ds(i * _L, _L)]
        l2c_lo = lambda i: h2c[pl.ds(off_lo + i * _L, _L)]
        l2s_lo = lambda i: h2s[pl.ds(off_lo + i * _L, _L)]
        b2_hi, p2h, sp2h, c2h, s2h = _walk(
            l2c_hi, l2s_hi, _NB2 // _L, c_hi - rk_hi)
        b2_lo, p2l, sp2l, c2l, s2l = _walk(
            l2c_lo, l2s_lo, _NB2 // _L, rk_lo - 1)

        top_inc = top_inc + (s_hi - sp2h - s2h)
        rk_hi = rk_hi - (c_hi - p2h - c2h)
        bot_inc = bot_inc + sp2l
        rk_lo = rk_lo - p2l

        pref_hi = b1_hi * _NB2 + b2_hi
        pref_lo = b1_lo * _NB2 + b2_lo
        same2 = pref_hi == pref_lo

        def p3(j, c):
            v = row_v[pl.ds(j * _L, _L)]
            p = plsc.bitcast(v, jnp.int32)
            p21 = p >> _B3_BITS
            in_hi = p21 == pref_hi
            in_lo = p21 == pref_lo
            b3 = p & (_NB3 - 1)
            use_off = jnp.logical_and(in_lo, jnp.logical_not(same2))
            idx = b3 + jnp.where(use_off, _NB3, 0)
            msk = jnp.logical_or(in_hi, in_lo)
            plsc.addupdate_scatter(h3c, [idx], ones_i, mask=msk)
            return c
        lax.fori_loop(0, _T // _L, p3, 0)

        off3_lo = jnp.where(same2, 0, _NB3)

        def vals_of(pref):
            return lambda i: plsc.bitcast(
                pref * _NB3 + i * _L + iota, jnp.float32)

        l3c_hi = lambda i: h3c[pl.ds(i * _L, _L)]
        l3c_lo = lambda i: h3c[pl.ds(off3_lo + i * _L, _L)]
        vhi = vals_of(pref_hi)
        vlo = vals_of(pref_lo)
        l3s_hi = lambda i: l3c_hi(i).astype(jnp.float32) * vhi(i)
        l3s_lo = lambda i: l3c_lo(i).astype(jnp.float32) * vlo(i)

        b3_hi, p3h, sp3h, c3h, s3h = _walk(
            l3c_hi, l3s_hi, _NB3 // _L, c2h - rk_hi)
        b3_lo, p3l, sp3l, c3l, s3l = _walk(
            l3c_lo, l3s_lo, _NB3 // _L, rk_lo - 1)

        theta_hi = plsc.bitcast(
            jnp.broadcast_to(pref_hi * _NB3 + b3_hi, (_L,)), jnp.float32)[0]
        theta_lo = plsc.bitcast(
            jnp.broadcast_to(pref_lo * _NB3 + b3_lo, (_L,)), jnp.float32)[0]

        top_inc = top_inc + (s2h - sp3h - s3h)
        rk_hi = rk_hi - (c2h - p3h - c3h)
        bot_inc = bot_inc + sp3l
        rk_lo = rk_lo - p3l

        top_sum = top_inc + rk_hi.astype(jnp.float32) * theta_hi
        bot_sum = bot_inc + rk_lo.astype(jnp.float32) * theta_lo
        return acc + (bot_sum - top_sum)

    acc = lax.fori_loop(0, _ROWS_PER_W, row_body, jnp.float32(0.0))
    outbuf[...] = jnp.where(iota == 0, acc, jnp.float32(0.0))
    pltpu.sync_copy(outbuf, out_hbm.at[wid])


def kernel(att):
    n, t = att.shape
    k = max(t // 8, 1)
    mesh = plsc.VectorSubcoreMesh(core_axis_name="c", subcore_axis_name="s", num_cores=_NC, num_subcores=_NS)
    partials = pl.kernel(
        _sc_loss_kernel,
        out_type=jax.ShapeDtypeStruct((_NW, _L), jnp.float32),
        mesh=mesh,
        compiler_params=pltpu.CompilerParams(needs_layout_passes=False),
        scratch_types=[
            pltpu.VMEM((_T,), jnp.float32),
            pltpu.VMEM((_NB1,), jnp.int32),
            pltpu.VMEM((_NB1,), jnp.float32),
            pltpu.VMEM((2 * _NB2,), jnp.int32),
            pltpu.VMEM((2 * _NB2,), jnp.float32),
            pltpu.VMEM((2 * _NB3,), jnp.int32),
            pltpu.VMEM((_L,), jnp.float32),
        ],
    )(att)
    return (jnp.sum(partials) / (k * n)).astype(jnp.float32)


# speedup vs baseline: 2.6320x; 2.6320x over previous
import jax
import jax.numpy as jnp
from jax import lax
from jax.experimental import pallas as pl
from jax.experimental.pallas import tpu as pltpu
from jax.experimental.pallas import tpu_sc as plsc

_NC = 2
_NS = 16
_NW = _NC * _NS
_L = 16

_N_ROWS = 128
_T = 32768
_K = _T // 8
_ROWS_PER_W = _N_ROWS // _NW

_B_BITS = 10
_NB = 1 << _B_BITS
_UNROLL = 8
_WALK_UNROLL = 4


def _walk(loadc, loads, nchunks, tau):
    iota = lax.iota(jnp.int32, _L)

    z = (jnp.int32(0), jnp.int32(0), jnp.float32(0.0), jnp.int32(0),
         jnp.int32(0), jnp.float32(0.0), jnp.int32(0), jnp.float32(0.0))

    @plsc.parallel_loop(0, nchunks, unroll=_WALK_UNROLL, carry=z)
    def body(i, carry):
        found, run_c, run_s, bst, pst, spst, cst, sst = carry
        cc = loadc(i)
        ss = loads(i)
        tot_c = jnp.sum(cc)
        tot_s = jnp.sum(ss)
        cross = jnp.logical_and(found == 0, run_c + tot_c > tau)

        def heavy(op):
            cc, ss, run_c, run_s, _f, _b, _p, _sp, _c, _s = op
            ic = plsc.cumsum(cc)
            p_lane = run_c + ic - cc
            m = jnp.logical_and(p_lane <= tau, tau < p_lane + cc)
            fs = plsc.cumsum(ss)
            sp_lane = run_s + fs - ss
            zi = jnp.zeros((_L,), jnp.int32)
            zf = jnp.zeros((_L,), jnp.float32)
            b = i * _L + jnp.sum(jnp.where(m, iota, zi))
            p = jnp.sum(jnp.where(m, p_lane, zi))
            sp = jnp.sum(jnp.where(m, sp_lane, zf))
            c = jnp.sum(jnp.where(m, cc, zi))
            s = jnp.sum(jnp.where(m, ss, zf))
            return (jnp.int32(1), b, p, sp, c, s)

        def light(op):
            _cc, _ss, _rc, _rs, f, b, p, sp, c, s = op
            return (f, b, p, sp, c, s)

        found, bst, pst, spst, cst, sst = lax.cond(
            cross, heavy, light,
            (cc, ss, run_c, run_s, found, bst, pst, spst, cst, sst))
        return (found, run_c + tot_c, run_s + tot_s, bst, pst, spst, cst, sst)

    return body[3], body[4], body[5], body[6], body[7]


def _sc_loss_kernel(att_hbm, out_hbm, row_v, cnt1, sum1, h2c, h2s, h3c,
                    outbuf):
    wid = lax.axis_index("s") * _NC + lax.axis_index("c")
    iota = lax.iota(jnp.int32, _L)
    ones_i = jnp.ones((_L,), jnp.int32)
    zeros_i = jnp.zeros((_L,), jnp.int32)
    zeros_f = jnp.zeros((_L,), jnp.float32)
    kk = jnp.int32(_K)

    def row_body(r, acc):
        row = wid * _ROWS_PER_W + r
        pltpu.sync_copy(att_hbm.at[row], row_v)

        @plsc.parallel_loop(0, _NB // _L, unroll=_UNROLL)
        def _z1(i):
            cnt1[pl.ds(i * _L, _L)] = zeros_i
            sum1[pl.ds(i * _L, _L)] = zeros_f

        @plsc.parallel_loop(0, (2 * _NB) // _L, unroll=_UNROLL)
        def _z2(i):
            h2c[pl.ds(i * _L, _L)] = zeros_i
            h2s[pl.ds(i * _L, _L)] = zeros_f
            h3c[pl.ds(i * _L, _L)] = zeros_i

        @plsc.parallel_loop(0, _T // _L, unroll=_UNROLL, carry=zeros_f)
        def p1(j, sacc):
            v = row_v[pl.ds(j * _L, _L)]
            p = plsc.bitcast(v, jnp.int32)
            b1 = p >> (30 - _B_BITS)
            plsc.addupdate_scatter(cnt1, [b1], ones_i)
            plsc.addupdate_scatter(sum1, [b1], v)
            return sacc + v
        total_sum = jnp.sum(p1)

        l1c = lambda i: cnt1[pl.ds(i * _L, _L)]
        l1s = lambda i: sum1[pl.ds(i * _L, _L)]
        b1_hi, p_hi, sp_hi, c_hi, s_hi = _walk(
            l1c, l1s, _NB // _L, jnp.int32(_T - _K))
        b1_lo, p_lo, sp_lo, c_lo, s_lo = _walk(
            l1c, l1s, _NB // _L, jnp.int32(_K - 1))

        top_inc = total_sum - sp_hi - s_hi
        rk_hi = kk - (jnp.int32(_T) - p_hi - c_hi)
        bot_inc = sp_lo
        rk_lo = kk - p_lo
        same1 = b1_hi == b1_lo

        @plsc.parallel_loop(0, _T // _L, unroll=_UNROLL)
        def _p2(j):
            v = row_v[pl.ds(j * _L, _L)]
            p = plsc.bitcast(v, jnp.int32)
            b1 = p >> (30 - _B_BITS)
            in_hi = b1 == b1_hi
            in_lo = b1 == b1_lo
            b2 = (p >> _B_BITS) & (_NB - 1)
            use_off = jnp.logical_and(in_lo, jnp.logical_not(same1))
            idx = b2 + jnp.where(use_off, _NB, 0)
            msk = jnp.logical_or(in_hi, in_lo)
            plsc.addupdate_scatter(h2c, [idx], ones_i, mask=msk)
            plsc.addupdate_scatter(h2s, [idx], v, mask=msk)

        off_lo = jnp.where(same1, 0, _NB)
        l2c_hi = lambda i: h2c[pl.ds(i * _L, _L)]
        l2s_hi = lambda i: h2s[pl.ds(i * _L, _L)]
        l2c_lo = lambda i: h2c[pl.ds(off_lo + i * _L, _L)]
        l2s_lo = lambda i: h2s[pl.ds(off_lo + i * _L, _L)]
        b2_hi, p2h, sp2h, c2h, s2h = _walk(
            l2c_hi, l2s_hi, _NB // _L, c_hi - rk_hi)
        b2_lo, p2l, sp2l, c2l, s2l = _walk(
            l2c_lo, l2s_lo, _NB // _L, rk_lo - 1)

        top_inc = top_inc + (s_hi - sp2h - s2h)
        rk_hi = rk_hi - (c_hi - p2h - c2h)
        bot_inc = bot_inc + sp2l
        rk_lo = rk_lo - p2l

        pref_hi = b1_hi * _NB + b2_hi
        pref_lo = b1_lo * _NB + b2_lo
        same2 = pref_hi == pref_lo

        @plsc.parallel_loop(0, _T // _L, unroll=_UNROLL)
        def _p3(j):
            v = row_v[pl.ds(j * _L, _L)]
            p = plsc.bitcast(v, jnp.int32)
            p20 = p >> _B_BITS
            in_hi = p20 == pref_hi
            in_lo = p20 == pref_lo
            b3 = p & (_NB - 1)
            use_off = jnp.logical_and(in_lo, jnp.logical_not(same2))
            idx = b3 + jnp.where(use_off, _NB, 0)
            msk = jnp.logical_or(in_hi, in_lo)
            plsc.addupdate_scatter(h3c, [idx], ones_i, mask=msk)

        off3_lo = jnp.where(same2, 0, _NB)

        def vals_of(pref):
            return lambda i: plsc.bitcast(
                pref * _NB + i * _L + iota, jnp.float32)

        l3c_hi = lambda i: h3c[pl.ds(i * _L, _L)]
        l3c_lo = lambda i: h3c[pl.ds(off3_lo + i * _L, _L)]
        vhi = vals_of(pref_hi)
        vlo = vals_of(pref_lo)
        l3s_hi = lambda i: l3c_hi(i).astype(jnp.float32) * vhi(i)
        l3s_lo = lambda i: l3c_lo(i).astype(jnp.float32) * vlo(i)

        b3_hi, p3h, sp3h, c3h, s3h = _walk(
            l3c_hi, l3s_hi, _NB // _L, c2h - rk_hi)
        b3_lo, p3l, sp3l, c3l, s3l = _walk(
            l3c_lo, l3s_lo, _NB // _L, rk_lo - 1)

        theta_hi = plsc.bitcast(
            jnp.broadcast_to(pref_hi * _NB + b3_hi, (_L,)), jnp.float32)[0]
        theta_lo = plsc.bitcast(
            jnp.broadcast_to(pref_lo * _NB + b3_lo, (_L,)), jnp.float32)[0]

        top_inc = top_inc + (s2h - sp3h - s3h)
        rk_hi = rk_hi - (c2h - p3h - c3h)
        bot_inc = bot_inc + sp3l
        rk_lo = rk_lo - p3l

        top_sum = top_inc + rk_hi.astype(jnp.float32) * theta_hi
        bot_sum = bot_inc + rk_lo.astype(jnp.float32) * theta_lo
        return acc + (bot_sum - top_sum)

    acc = lax.fori_loop(0, _ROWS_PER_W, row_body, jnp.float32(0.0))
    outbuf[...] = jnp.where(iota == 0, acc, jnp.float32(0.0))
    pltpu.sync_copy(outbuf, out_hbm.at[wid])


def kernel(att):
    n, t = att.shape
    k = max(t // 8, 1)
    mesh = plsc.VectorSubcoreMesh(core_axis_name="c", subcore_axis_name="s",
                                  num_cores=_NC, num_subcores=_NS)
    partials = pl.kernel(
        _sc_loss_kernel,
        out_type=jax.ShapeDtypeStruct((_NW, _L), jnp.float32),
        mesh=mesh,
        compiler_params=pltpu.CompilerParams(needs_layout_passes=False),
        scratch_types=[
            pltpu.VMEM((_T,), jnp.float32),
            pltpu.VMEM((_NB,), jnp.int32),
            pltpu.VMEM((_NB,), jnp.float32),
            pltpu.VMEM((2 * _NB,), jnp.int32),
            pltpu.VMEM((2 * _NB,), jnp.float32),
            pltpu.VMEM((2 * _NB,), jnp.int32),
            pltpu.VMEM((_L,), jnp.float32),
        ],
    )(att)
    return (jnp.sum(partials) / (k * n)).astype(jnp.float32)


# speedup vs baseline: 2.9502x; 1.1209x over previous
import jax
import jax.numpy as jnp
from jax import lax
from jax.experimental import pallas as pl
from jax.experimental.pallas import tpu as pltpu
from jax.experimental.pallas import tpu_sc as plsc

_NC = 2
_NS = 16
_NW = _NC * _NS
_L = 16

_N_ROWS = 128
_T = 32768
_K = _T // 8
_ROWS_PER_W = _N_ROWS // _NW

_B_BITS = 10
_NB = 1 << _B_BITS
_UNROLL = 8
_WALK_UNROLL = 4


def _walk(loadc, nchunks, tau):
    iota = lax.iota(jnp.int32, _L)

    z = (jnp.int32(0), jnp.int32(0), jnp.int32(0), jnp.int32(0),
         jnp.int32(0))

    @plsc.parallel_loop(0, nchunks, unroll=_WALK_UNROLL, carry=z)
    def body(i, carry):
        found, run_c, bst, pst, cst = carry
        cc = loadc(i)
        tot_c = jnp.sum(cc)
        cross = jnp.logical_and(found == 0, run_c + tot_c > tau)

        def heavy(op):
            cc, run_c, _f, _b, _p, _c = op
            ic = plsc.cumsum(cc)
            p_lane = run_c + ic - cc
            m = jnp.logical_and(p_lane <= tau, tau < p_lane + cc)
            zi = jnp.zeros((_L,), jnp.int32)
            b = i * _L + jnp.sum(jnp.where(m, iota, zi))
            p = jnp.sum(jnp.where(m, p_lane, zi))
            c = jnp.sum(jnp.where(m, cc, zi))
            return (jnp.int32(1), b, p, c)

        def light(op):
            _cc, _rc, f, b, p, c = op
            return (f, b, p, c)

        found, bst, pst, cst = lax.cond(
            cross, heavy, light, (cc, run_c, found, bst, pst, cst))
        return (found, run_c + tot_c, bst, pst, cst)

    return body[2], body[3], body[4]


def _process_row(row_v, h1, h2, h3):
    zeros_i = jnp.zeros((_L,), jnp.int32)
    zeros_f = jnp.zeros((_L,), jnp.float32)
    ones_i = jnp.ones((_L,), jnp.int32)
    kk = jnp.int32(_K)

    @plsc.parallel_loop(0, _NB // _L, unroll=_UNROLL)
    def _z1(i):
        h1[pl.ds(i * _L, _L)] = zeros_i

    @plsc.parallel_loop(0, (2 * _NB) // _L, unroll=_UNROLL)
    def _z2(i):
        h2[pl.ds(i * _L, _L)] = zeros_i
        h3[pl.ds(i * _L, _L)] = zeros_i

    @plsc.parallel_loop(0, _T // _L, unroll=_UNROLL)
    def _p1(j):
        v = row_v[pl.ds(j * _L, _L)]
        p = plsc.bitcast(v, jnp.int32)
        b1 = p >> (30 - _B_BITS)
        plsc.addupdate_scatter(h1, [b1], ones_i)

    l1c = lambda i: h1[pl.ds(i * _L, _L)]
    b1_hi, p_hi, c_hi = _walk(l1c, _NB // _L, jnp.int32(_T - _K))
    b1_lo, p_lo, c_lo = _walk(l1c, _NB // _L, jnp.int32(_K - 1))
    rk_hi = kk - (jnp.int32(_T) - p_hi - c_hi)
    rk_lo = kk - p_lo
    same1 = b1_hi == b1_lo

    @plsc.parallel_loop(0, _T // _L, unroll=_UNROLL)
    def _p2(j):
        v = row_v[pl.ds(j * _L, _L)]
        p = plsc.bitcast(v, jnp.int32)
        b1 = p >> (30 - _B_BITS)
        in_hi = b1 == b1_hi
        in_lo = b1 == b1_lo
        b2 = (p >> _B_BITS) & (_NB - 1)
        use_off = jnp.logical_and(in_lo, jnp.logical_not(same1))
        idx = b2 + jnp.where(use_off, _NB, 0)
        msk = jnp.logical_or(in_hi, in_lo)
        plsc.addupdate_scatter(h2, [idx], ones_i, mask=msk)

    off_lo = jnp.where(same1, 0, _NB)
    l2c_hi = lambda i: h2[pl.ds(i * _L, _L)]
    l2c_lo = lambda i: h2[pl.ds(off_lo + i * _L, _L)]
    b2_hi, p2h, c2h = _walk(l2c_hi, _NB // _L, c_hi - rk_hi)
    b2_lo, p2l, c2l = _walk(l2c_lo, _NB // _L, rk_lo - 1)
    rk_hi = rk_hi - (c_hi - p2h - c2h)
    rk_lo = rk_lo - p2l

    pref_hi = b1_hi * _NB + b2_hi
    pref_lo = b1_lo * _NB + b2_lo
    same2 = pref_hi == pref_lo

    @plsc.parallel_loop(0, _T // _L, unroll=_UNROLL)
    def _p3(j):
        v = row_v[pl.ds(j * _L, _L)]
        p = plsc.bitcast(v, jnp.int32)
        p20 = p >> _B_BITS
        in_hi = p20 == pref_hi
        in_lo = p20 == pref_lo
        b3 = p & (_NB - 1)
        use_off = jnp.logical_and(in_lo, jnp.logical_not(same2))
        idx = b3 + jnp.where(use_off, _NB, 0)
        msk = jnp.logical_or(in_hi, in_lo)
        plsc.addupdate_scatter(h3, [idx], ones_i, mask=msk)

    off3_lo = jnp.where(same2, 0, _NB)
    l3c_hi = lambda i: h3[pl.ds(i * _L, _L)]
    l3c_lo = lambda i: h3[pl.ds(off3_lo + i * _L, _L)]
    b3_hi, p3h, c3h = _walk(l3c_hi, _NB // _L, c2h - rk_hi)
    b3_lo, p3l, c3l = _walk(l3c_lo, _NB // _L, rk_lo - 1)
    rk_hi = rk_hi - (c2h - p3h - c3h)
    rk_lo = rk_lo - p3l

    theta_hi_v = plsc.bitcast(
        jnp.broadcast_to(pref_hi * _NB + b3_hi, (_L,)), jnp.float32)
    theta_lo_v = plsc.bitcast(
        jnp.broadcast_to(pref_lo * _NB + b3_lo, (_L,)), jnp.float32)

    @plsc.parallel_loop(0, _T // _L, unroll=_UNROLL,
                        carry=(zeros_f, zeros_f))
    def sums(j, carry):
        sg, sl = carry
        v = row_v[pl.ds(j * _L, _L)]
        sg = sg + jnp.where(v > theta_hi_v, v, zeros_f)
        sl = sl + jnp.where(v < theta_lo_v, v, zeros_f)
        return (sg, sl)

    top_sum = jnp.sum(sums[0]) + rk_hi.astype(jnp.float32) * theta_hi_v[0]
    bot_sum = jnp.sum(sums[1]) + rk_lo.astype(jnp.float32) * theta_lo_v[0]
    return bot_sum - top_sum


def _sc_loss_kernel(att_hbm, out_hbm, row_a, row_b, h1, h2, h3, outbuf,
                    sem_a, sem_b):
    wid = lax.axis_index("s") * _NC + lax.axis_index("c")
    iota = lax.iota(jnp.int32, _L)
    base = wid * _ROWS_PER_W

    bufs = [(row_a, sem_a), (row_b, sem_b)]
    cps = [None, None]
    cps[0] = pltpu.async_copy(att_hbm.at[base], row_a, sem_a)
    acc = jnp.float32(0.0)
    for r in range(_ROWS_PER_W):
        buf, _ = bufs[r % 2]
        cps[r % 2].wait()
        if r + 1 < _ROWS_PER_W:
            nbuf, nsem = bufs[(r + 1) % 2]
            cps[(r + 1) % 2] = pltpu.async_copy(
                att_hbm.at[base + r + 1], nbuf, nsem)
        acc = acc + _process_row(buf, h1, h2, h3)

    outbuf[...] = jnp.where(iota == 0, acc, jnp.float32(0.0))
    pltpu.sync_copy(outbuf, out_hbm.at[wid])


def kernel(att):
    n, t = att.shape
    k = max(t // 8, 1)
    mesh = plsc.VectorSubcoreMesh(core_axis_name="c", subcore_axis_name="s",
                                  num_cores=_NC, num_subcores=_NS)
    partials = pl.kernel(
        _sc_loss_kernel,
        out_type=jax.ShapeDtypeStruct((_NW, _L), jnp.float32),
        mesh=mesh,
        compiler_params=pltpu.CompilerParams(needs_layout_passes=False),
        scratch_types=[
            pltpu.VMEM((_T,), jnp.float32),
            pltpu.VMEM((_T,), jnp.float32),
            pltpu.VMEM((_NB,), jnp.int32),
            pltpu.VMEM((2 * _NB,), jnp.int32),
            pltpu.VMEM((2 * _NB,), jnp.int32),
            pltpu.VMEM((_L,), jnp.float32),
            pltpu.SemaphoreType.DMA,
            pltpu.SemaphoreType.DMA,
        ],
    )(att)
    return (jnp.sum(partials) / (k * n)).astype(jnp.float32)
